# baseline (speedup 1.0000x reference)
import functools

import jax
import jax.numpy as jnp
from jax import lax
from jax.experimental import pallas as pl
from jax.experimental.pallas import tpu as pltpu
from jax.experimental.pallas import tpu_sc as plsc

_NC = 2
_NS = 16
_NPAD = 10240
_ROWS = 2000
_KA = 104
_NBUF = 3


def _sc_mesh():
    return plsc.VectorSubcoreMesh(core_axis_name="c", subcore_axis_name="s")


_SC_PARAMS = pltpu.CompilerParams(use_tc_tiling_on_sc=False,
                                  needs_layout_passes=False)


def _sc_degrees(idx4, z_pad, iota_pad):
    four, E = idx4.shape
    epw = E // (_NC * _NS)
    epw_pad = ((epw + 15) // 16) * 16
    nvec = epw_pad // 16
    cpt = _NPAD // _NS

    @functools.partial(
        pl.kernel,
        out_type=jax.ShapeDtypeStruct((4, _NC, _NPAD), jnp.float32),
        mesh=_sc_mesh(),
        compiler_params=_SC_PARAMS,
        scratch_types=[
            pltpu.VMEM_SHARED((_NPAD,), jnp.float32),
            pltpu.VMEM((_NPAD,), jnp.float32),
            pltpu.VMEM((_NPAD,), jnp.int32),
            pltpu.VMEM((epw_pad,), jnp.int32),
        ],
    )
    def k(idx_hbm, z_hbm, iota_hbm, out_hbm, acc, lhist, iotabuf, ibuf):
        c = lax.axis_index("c")
        s = lax.axis_index("s")
        wid = c * _NS + s
        ones = jnp.full((16,), 1.0, jnp.float32)
        lanes = lax.iota(jnp.int32, 16)
        pltpu.sync_copy(iota_hbm, iotabuf)
        for a in range(4):
            pltpu.sync_copy(z_hbm.at[pl.ds(0, cpt)], acc.at[pl.ds(s * cpt, cpt)])
            pltpu.sync_copy(z_hbm, lhist)
            pltpu.sync_copy(idx_hbm.at[a, pl.ds(wid * epw, epw)],
                            ibuf.at[pl.ds(0, epw)])

            @pl.loop(0, nvec)
            def _(j):
                iv = ibuf[pl.ds(j * 16, 16)]
                m = (j * 16 + lanes) < epw
                plsc.addupdate_scatter(lhist, [iv], ones, mask=m)

            plsc.subcore_barrier()
            pltpu.sync_copy(lhist, acc.at[iotabuf], add=True)
            plsc.subcore_barrier()
            pltpu.sync_copy(acc.at[pl.ds(s * cpt, cpt)],
                            out_hbm.at[a, c, pl.ds(s * cpt, cpt)])
            plsc.subcore_barrier()

    return k(idx4, z_pad, iota_pad)


def _sc_agg(h, srcm, srct, dstm, dstt, z_rows):
    N, D = h.shape
    _, nchunks, K = srcm.shape
    tail = srct.shape[1]
    rpt = N // _NS
    nb = _NBUF
    assert nchunks % nb == 0 and nchunks >= 2 * nb

    @functools.partial(
        pl.kernel,
        out_type=jax.ShapeDtypeStruct((_NC * N, D), jnp.bfloat16),
        mesh=_sc_mesh(),
        compiler_params=_SC_PARAMS,
        scratch_types=(
            [pltpu.VMEM_SHARED((N, D), jnp.bfloat16)]
            + [pltpu.VMEM((K, D), jnp.bfloat16)] * nb
            + [pltpu.VMEM((nchunks, K), jnp.int32),
               pltpu.VMEM((nchunks, K), jnp.int32),
               pltpu.VMEM((tail,), jnp.int32),
               pltpu.VMEM((tail,), jnp.int32)]
            + [pltpu.SemaphoreType.DMA] * (2 * nb)
        ),
    )
    def k(h_hbm, srcm_hbm, srct_hbm, dstm_hbm, dstt_hbm, z_hbm, out_hbm,
          acc, *rest):
        gbufs = list(rest[:nb])
        siall, diall, sit, dit = rest[nb:nb + 4]
        gsems = list(rest[nb + 4:nb + 4 + nb])
        ssems = list(rest[nb + 4 + nb:])
        c = lax.axis_index("c")
        s = lax.axis_index("s")
        w = c * _NS + s
        rbase = s * rpt
        pltpu.sync_copy(srcm_hbm.at[w], siall)
        pltpu.sync_copy(dstm_hbm.at[w], diall)
        pltpu.sync_copy(srct_hbm.at[w], sit)
        pltpu.sync_copy(dstt_hbm.at[w], dit)
        pltpu.sync_copy(z_hbm, acc.at[pl.ds(rbase, rpt)])
        plsc.subcore_barrier()

        def g_start(j, g):
            pltpu.async_copy(h_hbm.at[siall.at[g]], gbufs[j], gsems[j])

        def g_wait(j, g):
            pltpu.make_async_copy(h_hbm.at[siall.at[g]], gbufs[j],
                                  gsems[j]).wait()

        def s_start(j, g):
            pltpu.async_copy(gbufs[j], acc.at[diall.at[g]], ssems[j],
                             add=True)

        def s_wait(j, g):
            pltpu.make_async_copy(gbufs[j], acc.at[diall.at[g]],
                                  ssems[j]).wait()

        def step(j, x, first, start_next):
            g_wait(j, x)
            s_start(j, x)
            if not first:
                s_wait((j + nb - 1) % nb, x - 1)
            if start_next:
                g_start((j + 2) % nb, x + 2)

        for j in range(nb - 1):
            g_start(j, j)
        for j in range(nb):
            step(j, j, first=(j == 0), start_next=True)

        @pl.loop(1, nchunks // nb - 1)
        def _(t):
            x0 = t * nb
            for j in range(nb):
                step(j, x0 + j, first=False, start_next=True)

        x0 = nchunks - nb
        for j in range(nb):
            step(j, x0 + j, first=False, start_next=(x0 + j + 2 < nchunks))
        s_wait((nchunks - 1) % nb, nchunks - 1)
        tslc = pl.ds(0, tail)
        pltpu.async_copy(h_hbm.at[sit], gbufs[0].at[tslc], gsems[0]).wait()
        pltpu.async_copy(gbufs[0].at[tslc], acc.at[dit], ssems[0],
                         add=True).wait()

        plsc.subcore_barrier()
        pltpu.sync_copy(acc.at[pl.ds(rbase, rpt)],
                        out_hbm.at[pl.ds(c * N + rbase, rpt)])

    return k(h, srcm, srct, dstm, dstt, z_rows)


def _prep_body(x_ref, do0_ref, do1_ref, di0_ref, di1_ref,
               h_ref, nin_ref, nout_ref):
    dout = do0_ref[...] + do1_ref[...]
    din = di0_ref[...] + di1_ref[...]
    nout = lax.rsqrt(jnp.maximum(dout, 1.0))
    nin = lax.rsqrt(jnp.maximum(din, 1.0))
    h_ref[...] = (x_ref[...] * nout).astype(jnp.bfloat16)
    nin_ref[...] = nin
    nout_ref[...] = nout


def _tc_prep(x, do0, do1, di0, di1):
    n, d = x.shape
    grid = (n // _ROWS,)
    vec = pl.BlockSpec((_ROWS, 1), lambda i: (i, 0))
    return pl.pallas_call(
        _prep_body,
        grid=grid,
        in_specs=[pl.BlockSpec((_ROWS, d), lambda i: (i, 0)), vec, vec, vec, vec],
        out_specs=[pl.BlockSpec((_ROWS, d), lambda i: (i, 0)), vec, vec],
        out_shape=[jax.ShapeDtypeStruct((n, d), jnp.bfloat16),
                   jax.ShapeDtypeStruct((n, 1), jnp.float32),
                   jax.ShapeDtypeStruct((n, 1), jnp.float32)],
    )(x, do0, do1, di0, di1)


def _layer_body(a0_ref, a1_ref, nin_ref, nout_ref, w_ref, b_ref, h_ref):
    a = (a0_ref[...].astype(jnp.float32) + a1_ref[...].astype(jnp.float32))
    a = a * nin_ref[...]
    z = jnp.dot(a.astype(jnp.bfloat16), w_ref[...].astype(jnp.bfloat16),
                preferred_element_type=jnp.float32) + b_ref[...]
    h_ref[...] = (jax.nn.sigmoid(z) * nout_ref[...]).astype(jnp.bfloat16)


def _tc_layer(agg2, nin, nout, W, b):
    n2, d = agg2.shape
    n = n2 // 2
    grid = (n // _ROWS,)
    nblk = n // _ROWS
    vec = pl.BlockSpec((_ROWS, 1), lambda i: (i, 0))
    return pl.pallas_call(
        _layer_body,
        grid=grid,
        in_specs=[pl.BlockSpec((_ROWS, d), lambda i: (i, 0)),
                  pl.BlockSpec((_ROWS, d), lambda i: (i + nblk, 0)),
                  vec, vec,
                  pl.BlockSpec((d, d), lambda i: (0, 0)),
                  pl.BlockSpec((1, d), lambda i: (0, 0))],
        out_specs=pl.BlockSpec((_ROWS, d), lambda i: (i, 0)),
        out_shape=jax.ShapeDtypeStruct((n, d), jnp.bfloat16),
    )(agg2, agg2, nin, nout, W, b.reshape(1, d))


def _final_body(a0_ref, a1_ref, nin_ref, w_ref, b_ref, wnc_ref, bnc_ref,
                npred_ref, ysum_ref):
    i = pl.program_id(0)
    a = (a0_ref[...].astype(jnp.float32) + a1_ref[...].astype(jnp.float32))
    a = a * nin_ref[...]
    z = jnp.dot(a.astype(jnp.bfloat16), w_ref[...].astype(jnp.bfloat16),
                preferred_element_type=jnp.float32) + b_ref[...]
    y = jax.nn.sigmoid(z)
    npred_ref[...] = jax.nn.sigmoid(
        jnp.sum(y * wnc_ref[...], axis=1, keepdims=True) + bnc_ref[...])

    @pl.when(i == 0)
    def _():
        ysum_ref[...] = jnp.zeros_like(ysum_ref)

    ysum_ref[...] += jnp.sum(y, axis=0, keepdims=True)


def _tc_final(agg2, nin, W, b, wncT, bnc):
    n2, d = agg2.shape
    n = n2 // 2
    grid = (n // _ROWS,)
    nblk = n // _ROWS
    vec = pl.BlockSpec((_ROWS, 1), lambda i: (i, 0))
    full = lambda shape: pl.BlockSpec(shape, lambda i: (0, 0))
    return pl.pallas_call(
        _final_body,
        grid=grid,
        in_specs=[pl.BlockSpec((_ROWS, d), lambda i: (i, 0)),
                  pl.BlockSpec((_ROWS, d), lambda i: (i + nblk, 0)),
                  vec, full((d, d)), full((1, d)),
                  full((1, d)), full((1, 1))],
        out_specs=[vec, full((1, d))],
        out_shape=[jax.ShapeDtypeStruct((n, 1), jnp.float32),
                   jax.ShapeDtypeStruct((1, d), jnp.float32)],
    )(agg2, agg2, nin, W, b.reshape(1, d), wncT, bnc.reshape(1, 1))


def _head_body(sp_ref, ss_ref, wrop_ref, brop_ref, wros_ref, bros_ref,
               wm1_ref, bm1_ref, wm2_ref, bm2_ref, o_ref, *, n_nodes):
    inv = 1.0 / n_nodes
    bf = jnp.bfloat16
    gp = jax.nn.sigmoid(
        jnp.dot((sp_ref[...] * inv).astype(bf), wrop_ref[...].astype(bf),
                preferred_element_type=jnp.float32) + brop_ref[...])
    gs = jax.nn.sigmoid(
        jnp.dot((ss_ref[...] * inv).astype(bf), wros_ref[...].astype(bf),
                preferred_element_type=jnp.float32) + bros_ref[...])
    cat = jnp.concatenate([gp, gs], axis=1)
    z1 = jnp.dot(cat.astype(bf), wm1_ref[...].astype(bf),
                 preferred_element_type=jnp.float32) + bm1_ref[...]
    z2 = jnp.sum(z1 * wm2_ref[...], axis=1, keepdims=True) + bm2_ref[...]
    o_ref[...] = jax.nn.sigmoid(z2)


def _tc_head(sp, ss, Wro_p, bro_p, Wro_s, bro_s, Wm1, bm1, Wm2, bm2, n_nodes):
    d = sp.shape[1]
    full = lambda shape: pl.BlockSpec(shape, lambda: (0,) * len(shape))
    return pl.pallas_call(
        functools.partial(_head_body, n_nodes=n_nodes),
        in_specs=[full((1, d)), full((1, d)),
                  full((d, d)), full((1, d)),
                  full((d, d)), full((1, d)),
                  full((2 * d, d)), full((1, d)),
                  full((1, d)), full((1, 1))],
        out_specs=full((1, 1)),
        out_shape=jax.ShapeDtypeStruct((1, 1), jnp.float32),
    )(sp, ss, Wro_p, bro_p.reshape(1, d), Wro_s, bro_s.reshape(1, d),
      Wm1, bm1.reshape(1, d), Wm2.reshape(1, d), bm2.reshape(1, 1))


def _split_edges(v, e):
    ept = e // (_NC * _NS)
    nchunks = (ept // _KA) // _NBUF * _NBUF
    main = nchunks * _KA
    vt = v.reshape(_NC * _NS, ept)
    return vt[:, :main].reshape(_NC * _NS, nchunks, _KA), vt[:, main:]


def _branch(x, src, dst, W, b, do0, do1, di0, di1, z_rows, Wnc, bnc):
    L = W.shape[0]
    e = src.shape[0]
    srcm, srct = _split_edges(src, e)
    dstm, dstt = _split_edges(dst, e)
    h, nin, nout = _tc_prep(x, do0, do1, di0, di1)
    for i in range(L - 1):
        agg2 = _sc_agg(h, srcm, srct, dstm, dstt, z_rows)
        h = _tc_layer(agg2, nin, nout, W[i], b[i])
    agg2 = _sc_agg(h, srcm, srct, dstm, dstt, z_rows)
    d = x.shape[1]
    npred, ysum = _tc_final(agg2, nin, W[L - 1], b[L - 1],
                            Wnc.reshape(1, d), bnc)
    return npred, ysum


def kernel(x_p, edge_index_p, x_s, edge_index_s, Wp, bp, Ws, bs,
           Wro_p, bro_p, Wro_s, bro_s, Wnc_p, bnc_p, Wnc_s, bnc_s,
           Wm1, bm1, Wm2, bm2):
    n, d = x_p.shape
    idx4 = jnp.concatenate([edge_index_p, edge_index_s], axis=0)
    z_pad = jnp.zeros((_NPAD,), jnp.float32)
    iota_pad = jnp.arange(_NPAD, dtype=jnp.int32)
    z_rows = jnp.zeros((n // _NS, d), jnp.bfloat16)

    deg = _sc_degrees(idx4, z_pad, iota_pad)
    dslc = lambda a, c: deg[a, c, :n, None]

    L = Wp.shape[0]
    e = edge_index_p.shape[1]
    spm, spt = _split_edges(edge_index_p[0], e)
    dpm, dpt = _split_edges(edge_index_p[1], e)
    ssm, sst = _split_edges(edge_index_s[0], e)
    dsm, dst_ = _split_edges(edge_index_s[1], e)
    hp, nin_p, nout_p = _tc_prep(x_p, dslc(0, 0), dslc(0, 1),
                                 dslc(1, 0), dslc(1, 1))
    hs, nin_s, nout_s = _tc_prep(x_s, dslc(2, 0), dslc(2, 1),
                                 dslc(3, 0), dslc(3, 1))
    for i in range(L - 1):
        ap = _sc_agg(hp, spm, spt, dpm, dpt, z_rows)
        a_s = _sc_agg(hs, ssm, sst, dsm, dst_, z_rows)
        hp = _tc_layer(ap, nin_p, nout_p, Wp[i], bp[i])
        hs = _tc_layer(a_s, nin_s, nout_s, Ws[i], bs[i])
    ap = _sc_agg(hp, spm, spt, dpm, dpt, z_rows)
    a_s = _sc_agg(hs, ssm, sst, dsm, dst_, z_rows)
    npred_p, ysum_p = _tc_final(ap, nin_p, Wp[L - 1], bp[L - 1],
                                Wnc_p.reshape(1, d), bnc_p)
    npred_s, ysum_s = _tc_final(a_s, nin_s, Ws[L - 1], bs[L - 1],
                                Wnc_s.reshape(1, d), bnc_s)

    dist = _tc_head(ysum_p, ysum_s, Wro_p, bro_p, Wro_s, bro_s,
                    Wm1, bm1, Wm2, bm2, n).reshape(1)
    return (dist, npred_p, npred_s)


# speedup vs baseline: 1.0044x; 1.0044x over previous
import functools

import jax
import jax.numpy as jnp
from jax import lax
from jax.experimental import pallas as pl
from jax.experimental.pallas import tpu as pltpu
from jax.experimental.pallas import tpu_sc as plsc

_NC = 2
_NS = 16
_NPAD = 10240
_ROWS = 2000
_KA = 104
_NBUF = 3


def _sc_mesh():
    return plsc.VectorSubcoreMesh(core_axis_name="c", subcore_axis_name="s")


_SC_PARAMS = pltpu.CompilerParams(use_tc_tiling_on_sc=False,
                                  needs_layout_passes=False)


def _sc_degrees(idx4, z_pad, iota_pad):
    four, E = idx4.shape
    epw = E // (_NC * _NS)
    epw_pad = ((epw + 15) // 16) * 16
    nvec = epw_pad // 16
    cpt = _NPAD // _NS

    @functools.partial(
        pl.kernel,
        out_type=jax.ShapeDtypeStruct((4, _NC, _NPAD), jnp.float32),
        mesh=_sc_mesh(),
        compiler_params=_SC_PARAMS,
        scratch_types=[
            pltpu.VMEM_SHARED((_NPAD,), jnp.float32),
            pltpu.VMEM((_NPAD,), jnp.float32),
            pltpu.VMEM((_NPAD,), jnp.int32),
            pltpu.VMEM((epw_pad,), jnp.int32),
        ],
    )
    def k(idx_hbm, z_hbm, iota_hbm, out_hbm, acc, lhist, iotabuf, ibuf):
        c = lax.axis_index("c")
        s = lax.axis_index("s")
        wid = c * _NS + s
        ones = jnp.full((16,), 1.0, jnp.float32)
        lanes = lax.iota(jnp.int32, 16)
        pltpu.sync_copy(iota_hbm, iotabuf)
        for a in range(4):
            pltpu.sync_copy(z_hbm.at[pl.ds(0, cpt)], acc.at[pl.ds(s * cpt, cpt)])
            pltpu.sync_copy(z_hbm, lhist)
            pltpu.sync_copy(idx_hbm.at[a, pl.ds(wid * epw, epw)],
                            ibuf.at[pl.ds(0, epw)])

            @pl.loop(0, nvec)
            def _(j):
                iv = ibuf[pl.ds(j * 16, 16)]
                m = (j * 16 + lanes) < epw
                plsc.addupdate_scatter(lhist, [iv], ones, mask=m)

            plsc.subcore_barrier()
            pltpu.sync_copy(lhist, acc.at[iotabuf], add=True)
            plsc.subcore_barrier()
            pltpu.sync_copy(acc.at[pl.ds(s * cpt, cpt)],
                            out_hbm.at[a, c, pl.ds(s * cpt, cpt)])
            plsc.subcore_barrier()

    return k(idx4, z_pad, iota_pad)


def _sc_agg(h, srcm, srct, dstm, dstt, z_rows):
    N, D = h.shape
    _, nchunks, K = srcm.shape
    tail = srct.shape[1]
    rpt = N // _NS
    nb = _NBUF
    assert nchunks % nb == 0 and nchunks >= 2 * nb

    @functools.partial(
        pl.kernel,
        out_type=jax.ShapeDtypeStruct((_NC * N, D), jnp.bfloat16),
        mesh=_sc_mesh(),
        compiler_params=_SC_PARAMS,
        scratch_types=(
            [pltpu.VMEM_SHARED((N, D), jnp.bfloat16)]
            + [pltpu.VMEM((K, D), jnp.bfloat16)] * nb
            + [pltpu.VMEM((nchunks, K), jnp.int32),
               pltpu.VMEM((nchunks, K), jnp.int32),
               pltpu.VMEM((tail,), jnp.int32),
               pltpu.VMEM((tail,), jnp.int32)]
            + [pltpu.SemaphoreType.DMA] * (2 * nb)
        ),
    )
    def k(h_hbm, srcm_hbm, srct_hbm, dstm_hbm, dstt_hbm, z_hbm, out_hbm,
          acc, *rest):
        gbufs = list(rest[:nb])
        siall, diall, sit, dit = rest[nb:nb + 4]
        gsems = list(rest[nb + 4:nb + 4 + nb])
        ssems = list(rest[nb + 4 + nb:])
        c = lax.axis_index("c")
        s = lax.axis_index("s")
        w = c * _NS + s
        rbase = s * rpt
        def g_start(j, g):
            pltpu.async_copy(h_hbm.at[siall.at[g]], gbufs[j], gsems[j])

        def g_wait(j, g):
            pltpu.make_async_copy(h_hbm.at[siall.at[g]], gbufs[j],
                                  gsems[j]).wait()

        def s_start(j, g):
            pltpu.async_copy(gbufs[j], acc.at[diall.at[g]], ssems[j],
                             add=True)

        def s_wait(j, g):
            pltpu.make_async_copy(gbufs[j], acc.at[diall.at[g]],
                                  ssems[j]).wait()

        def step(j, x, first, start_next):
            g_wait(j, x)
            s_start(j, x)
            if not first:
                s_wait((j + nb - 1) % nb, x - 1)
            if start_next:
                g_start((j + 2) % nb, x + 2)

        pltpu.sync_copy(srcm_hbm.at[w], siall)
        for j in range(nb - 1):
            g_start(j, j)
        pltpu.sync_copy(dstm_hbm.at[w], diall)
        pltpu.sync_copy(srct_hbm.at[w], sit)
        pltpu.sync_copy(dstt_hbm.at[w], dit)
        pltpu.sync_copy(z_hbm, acc.at[pl.ds(rbase, rpt)])
        plsc.subcore_barrier()

        for j in range(nb):
            step(j, j, first=(j == 0), start_next=True)

        @pl.loop(1, nchunks // nb - 1)
        def _(t):
            x0 = t * nb
            for j in range(nb):
                step(j, x0 + j, first=False, start_next=True)

        x0 = nchunks - nb
        for j in range(nb):
            step(j, x0 + j, first=False, start_next=(x0 + j + 2 < nchunks))
        s_wait((nchunks - 1) % nb, nchunks - 1)
        tslc = pl.ds(0, tail)
        pltpu.async_copy(h_hbm.at[sit], gbufs[0].at[tslc], gsems[0]).wait()
        pltpu.async_copy(gbufs[0].at[tslc], acc.at[dit], ssems[0],
                         add=True).wait()

        plsc.subcore_barrier()
        pltpu.sync_copy(acc.at[pl.ds(rbase, rpt)],
                        out_hbm.at[pl.ds(c * N + rbase, rpt)])

    return k(h, srcm, srct, dstm, dstt, z_rows)


def _prep_body(x_ref, do0_ref, do1_ref, di0_ref, di1_ref,
               h_ref, nin_ref, nout_ref):
    dout = do0_ref[...] + do1_ref[...]
    din = di0_ref[...] + di1_ref[...]
    nout = lax.rsqrt(jnp.maximum(dout, 1.0))
    nin = lax.rsqrt(jnp.maximum(din, 1.0))
    h_ref[...] = (x_ref[...] * nout).astype(jnp.bfloat16)
    nin_ref[...] = nin
    nout_ref[...] = nout


def _tc_prep(x, do0, do1, di0, di1):
    n, d = x.shape
    grid = (n // _ROWS,)
    vec = pl.BlockSpec((_ROWS, 1), lambda i: (i, 0))
    return pl.pallas_call(
        _prep_body,
        grid=grid,
        in_specs=[pl.BlockSpec((_ROWS, d), lambda i: (i, 0)), vec, vec, vec, vec],
        out_specs=[pl.BlockSpec((_ROWS, d), lambda i: (i, 0)), vec, vec],
        out_shape=[jax.ShapeDtypeStruct((n, d), jnp.bfloat16),
                   jax.ShapeDtypeStruct((n, 1), jnp.float32),
                   jax.ShapeDtypeStruct((n, 1), jnp.float32)],
    )(x, do0, do1, di0, di1)


def _layer_body(a0_ref, a1_ref, nin_ref, nout_ref, w_ref, b_ref, h_ref):
    a = (a0_ref[...].astype(jnp.float32) + a1_ref[...].astype(jnp.float32))
    a = a * nin_ref[...]
    z = jnp.dot(a.astype(jnp.bfloat16), w_ref[...].astype(jnp.bfloat16),
                preferred_element_type=jnp.float32) + b_ref[...]
    h_ref[...] = (jax.nn.sigmoid(z) * nout_ref[...]).astype(jnp.bfloat16)


def _tc_layer(agg2, nin, nout, W, b):
    n2, d = agg2.shape
    n = n2 // 2
    grid = (n // _ROWS,)
    nblk = n // _ROWS
    vec = pl.BlockSpec((_ROWS, 1), lambda i: (i, 0))
    return pl.pallas_call(
        _layer_body,
        grid=grid,
        in_specs=[pl.BlockSpec((_ROWS, d), lambda i: (i, 0)),
                  pl.BlockSpec((_ROWS, d), lambda i: (i + nblk, 0)),
                  vec, vec,
                  pl.BlockSpec((d, d), lambda i: (0, 0)),
                  pl.BlockSpec((1, d), lambda i: (0, 0))],
        out_specs=pl.BlockSpec((_ROWS, d), lambda i: (i, 0)),
        out_shape=jax.ShapeDtypeStruct((n, d), jnp.bfloat16),
    )(agg2, agg2, nin, nout, W, b.reshape(1, d))


def _final_body(a0_ref, a1_ref, nin_ref, w_ref, b_ref, wnc_ref, bnc_ref,
                npred_ref, ysum_ref):
    i = pl.program_id(0)
    a = (a0_ref[...].astype(jnp.float32) + a1_ref[...].astype(jnp.float32))
    a = a * nin_ref[...]
    z = jnp.dot(a.astype(jnp.bfloat16), w_ref[...].astype(jnp.bfloat16),
                preferred_element_type=jnp.float32) + b_ref[...]
    y = jax.nn.sigmoid(z)
    npred_ref[...] = jax.nn.sigmoid(
        jnp.sum(y * wnc_ref[...], axis=1, keepdims=True) + bnc_ref[...])

    @pl.when(i == 0)
    def _():
        ysum_ref[...] = jnp.zeros_like(ysum_ref)

    ysum_ref[...] += jnp.sum(y, axis=0, keepdims=True)


def _tc_final(agg2, nin, W, b, wncT, bnc):
    n2, d = agg2.shape
    n = n2 // 2
    grid = (n // _ROWS,)
    nblk = n // _ROWS
    vec = pl.BlockSpec((_ROWS, 1), lambda i: (i, 0))
    full = lambda shape: pl.BlockSpec(shape, lambda i: (0, 0))
    return pl.pallas_call(
        _final_body,
        grid=grid,
        in_specs=[pl.BlockSpec((_ROWS, d), lambda i: (i, 0)),
                  pl.BlockSpec((_ROWS, d), lambda i: (i + nblk, 0)),
                  vec, full((d, d)), full((1, d)),
                  full((1, d)), full((1, 1))],
        out_specs=[vec, full((1, d))],
        out_shape=[jax.ShapeDtypeStruct((n, 1), jnp.float32),
                   jax.ShapeDtypeStruct((1, d), jnp.float32)],
    )(agg2, agg2, nin, W, b.reshape(1, d), wncT, bnc.reshape(1, 1))


def _head_body(sp_ref, ss_ref, wrop_ref, brop_ref, wros_ref, bros_ref,
               wm1_ref, bm1_ref, wm2_ref, bm2_ref, o_ref, *, n_nodes):
    inv = 1.0 / n_nodes
    bf = jnp.bfloat16
    gp = jax.nn.sigmoid(
        jnp.dot((sp_ref[...] * inv).astype(bf), wrop_ref[...].astype(bf),
                preferred_element_type=jnp.float32) + brop_ref[...])
    gs = jax.nn.sigmoid(
        jnp.dot((ss_ref[...] * inv).astype(bf), wros_ref[...].astype(bf),
                preferred_element_type=jnp.float32) + bros_ref[...])
    cat = jnp.concatenate([gp, gs], axis=1)
    z1 = jnp.dot(cat.astype(bf), wm1_ref[...].astype(bf),
                 preferred_element_type=jnp.float32) + bm1_ref[...]
    z2 = jnp.sum(z1 * wm2_ref[...], axis=1, keepdims=True) + bm2_ref[...]
    o_ref[...] = jax.nn.sigmoid(z2)


def _tc_head(sp, ss, Wro_p, bro_p, Wro_s, bro_s, Wm1, bm1, Wm2, bm2, n_nodes):
    d = sp.shape[1]
    full = lambda shape: pl.BlockSpec(shape, lambda: (0,) * len(shape))
    return pl.pallas_call(
        functools.partial(_head_body, n_nodes=n_nodes),
        in_specs=[full((1, d)), full((1, d)),
                  full((d, d)), full((1, d)),
                  full((d, d)), full((1, d)),
                  full((2 * d, d)), full((1, d)),
                  full((1, d)), full((1, 1))],
        out_specs=full((1, 1)),
        out_shape=jax.ShapeDtypeStruct((1, 1), jnp.float32),
    )(sp, ss, Wro_p, bro_p.reshape(1, d), Wro_s, bro_s.reshape(1, d),
      Wm1, bm1.reshape(1, d), Wm2.reshape(1, d), bm2.reshape(1, 1))


def _split_edges(v, e):
    ept = e // (_NC * _NS)
    nchunks = (ept // _KA) // _NBUF * _NBUF
    main = nchunks * _KA
    vt = v.reshape(_NC * _NS, ept)
    return vt[:, :main].reshape(_NC * _NS, nchunks, _KA), vt[:, main:]


def _branch(x, src, dst, W, b, do0, do1, di0, di1, z_rows, Wnc, bnc):
    L = W.shape[0]
    e = src.shape[0]
    srcm, srct = _split_edges(src, e)
    dstm, dstt = _split_edges(dst, e)
    h, nin, nout = _tc_prep(x, do0, do1, di0, di1)
    for i in range(L - 1):
        agg2 = _sc_agg(h, srcm, srct, dstm, dstt, z_rows)
        h = _tc_layer(agg2, nin, nout, W[i], b[i])
    agg2 = _sc_agg(h, srcm, srct, dstm, dstt, z_rows)
    d = x.shape[1]
    npred, ysum = _tc_final(agg2, nin, W[L - 1], b[L - 1],
                            Wnc.reshape(1, d), bnc)
    return npred, ysum


def kernel(x_p, edge_index_p, x_s, edge_index_s, Wp, bp, Ws, bs,
           Wro_p, bro_p, Wro_s, bro_s, Wnc_p, bnc_p, Wnc_s, bnc_s,
           Wm1, bm1, Wm2, bm2):
    n, d = x_p.shape
    idx4 = jnp.concatenate([edge_index_p, edge_index_s], axis=0)
    z_pad = jnp.zeros((_NPAD,), jnp.float32)
    iota_pad = jnp.arange(_NPAD, dtype=jnp.int32)
    z_rows = jnp.zeros((n // _NS, d), jnp.bfloat16)

    deg = _sc_degrees(idx4, z_pad, iota_pad)
    dslc = lambda a, c: deg[a, c, :n, None]

    L = Wp.shape[0]
    e = edge_index_p.shape[1]
    spm, spt = _split_edges(edge_index_p[0], e)
    dpm, dpt = _split_edges(edge_index_p[1], e)
    ssm, sst = _split_edges(edge_index_s[0], e)
    dsm, dst_ = _split_edges(edge_index_s[1], e)
    hp, nin_p, nout_p = _tc_prep(x_p, dslc(0, 0), dslc(0, 1),
                                 dslc(1, 0), dslc(1, 1))
    hs, nin_s, nout_s = _tc_prep(x_s, dslc(2, 0), dslc(2, 1),
                                 dslc(3, 0), dslc(3, 1))
    for i in range(L - 1):
        ap = _sc_agg(hp, spm, spt, dpm, dpt, z_rows)
        a_s = _sc_agg(hs, ssm, sst, dsm, dst_, z_rows)
        hp = _tc_layer(ap, nin_p, nout_p, Wp[i], bp[i])
        hs = _tc_layer(a_s, nin_s, nout_s, Ws[i], bs[i])
    ap = _sc_agg(hp, spm, spt, dpm, dpt, z_rows)
    a_s = _sc_agg(hs, ssm, sst, dsm, dst_, z_rows)
    npred_p, ysum_p = _tc_final(ap, nin_p, Wp[L - 1], bp[L - 1],
                                Wnc_p.reshape(1, d), bnc_p)
    npred_s, ysum_s = _tc_final(a_s, nin_s, Ws[L - 1], bs[L - 1],
                                Wnc_s.reshape(1, d), bnc_s)

    dist = _tc_head(ysum_p, ysum_s, Wro_p, bro_p, Wro_s, bro_s,
                    Wm1, bm1, Wm2, bm2, n).reshape(1)
    return (dist, npred_p, npred_s)


# speedup vs baseline: 1.0052x; 1.0007x over previous
import functools

import jax
import jax.numpy as jnp
from jax import lax
from jax.experimental import pallas as pl
from jax.experimental.pallas import tpu as pltpu
from jax.experimental.pallas import tpu_sc as plsc

_NC = 2
_NS = 16
_NPAD = 10240
_ROWS = 2000
_KA = 104
_NBUF = 3


def _sc_mesh():
    return plsc.VectorSubcoreMesh(core_axis_name="c", subcore_axis_name="s")


_SC_PARAMS = pltpu.CompilerParams(use_tc_tiling_on_sc=False,
                                  needs_layout_passes=False)


def _sc_degrees(idx4, z_pad, iota_pad):
    four, E = idx4.shape
    epw = E // (_NC * _NS)
    epw_pad = ((epw + 15) // 16) * 16
    nvec = epw_pad // 16
    cpt = _NPAD // _NS

    @functools.partial(
        pl.kernel,
        out_type=jax.ShapeDtypeStruct((4, _NC, _NPAD), jnp.float32),
        mesh=_sc_mesh(),
        compiler_params=_SC_PARAMS,
        scratch_types=[
            pltpu.VMEM_SHARED((_NPAD,), jnp.float32),
            pltpu.VMEM((_NPAD,), jnp.float32),
            pltpu.VMEM((_NPAD,), jnp.int32),
            pltpu.VMEM((epw_pad,), jnp.int32),
        ],
    )
    def k(idx_hbm, z_hbm, iota_hbm, out_hbm, acc, lhist, iotabuf, ibuf):
        c = lax.axis_index("c")
        s = lax.axis_index("s")
        wid = c * _NS + s
        ones = jnp.full((16,), 1.0, jnp.float32)
        lanes = lax.iota(jnp.int32, 16)
        pltpu.sync_copy(iota_hbm, iotabuf)
        for a in range(4):
            pltpu.sync_copy(z_hbm.at[pl.ds(0, cpt)], acc.at[pl.ds(s * cpt, cpt)])
            pltpu.sync_copy(z_hbm, lhist)
            pltpu.sync_copy(idx_hbm.at[a, pl.ds(wid * epw, epw)],
                            ibuf.at[pl.ds(0, epw)])

            @pl.loop(0, nvec)
            def _(j):
                iv = ibuf[pl.ds(j * 16, 16)]
                m = (j * 16 + lanes) < epw
                plsc.addupdate_scatter(lhist, [iv], ones, mask=m)

            plsc.subcore_barrier()
            pltpu.sync_copy(lhist, acc.at[iotabuf], add=True)
            plsc.subcore_barrier()
            pltpu.sync_copy(acc.at[pl.ds(s * cpt, cpt)],
                            out_hbm.at[a, c, pl.ds(s * cpt, cpt)])
            plsc.subcore_barrier()

    return k(idx4, z_pad, iota_pad)


def _sc_agg(h, srcm, srct, dstm, dstt, z_rows):
    N, D = h.shape
    _, nchunks, K = srcm.shape
    tail = srct.shape[1]
    rpt = N // _NS
    nb = _NBUF
    assert nchunks % nb == 0 and nchunks >= 2 * nb

    @functools.partial(
        pl.kernel,
        out_type=jax.ShapeDtypeStruct((_NC * N, D), jnp.bfloat16),
        mesh=_sc_mesh(),
        compiler_params=_SC_PARAMS,
        scratch_types=(
            [pltpu.VMEM_SHARED((N, D), jnp.bfloat16)]
            + [pltpu.VMEM((K, D), jnp.bfloat16)] * nb
            + [pltpu.VMEM((nchunks, K), jnp.int32),
               pltpu.VMEM((nchunks, K), jnp.int32),
               pltpu.VMEM((tail,), jnp.int32),
               pltpu.VMEM((tail,), jnp.int32)]
            + [pltpu.SemaphoreType.DMA] * (2 * nb)
        ),
    )
    def k(h_hbm, srcm_hbm, srct_hbm, dstm_hbm, dstt_hbm, z_hbm, out_hbm,
          acc, *rest):
        gbufs = list(rest[:nb])
        siall, diall, sit, dit = rest[nb:nb + 4]
        gsems = list(rest[nb + 4:nb + 4 + nb])
        ssems = list(rest[nb + 4 + nb:])
        c = lax.axis_index("c")
        s = lax.axis_index("s")
        w = c * _NS + s
        rbase = s * rpt
        def g_start(j, g):
            pltpu.async_copy(h_hbm.at[siall.at[g]], gbufs[j], gsems[j])

        def g_wait(j, g):
            pltpu.make_async_copy(h_hbm.at[siall.at[g]], gbufs[j],
                                  gsems[j]).wait()

        def s_start(j, g):
            pltpu.async_copy(gbufs[j], acc.at[diall.at[g]], ssems[j],
                             add=True)

        def s_wait(j, g):
            pltpu.make_async_copy(gbufs[j], acc.at[diall.at[g]],
                                  ssems[j]).wait()

        def step(j, x, first, start_next):
            g_wait(j, x)
            s_start(j, x)
            if not first:
                s_wait((j + nb - 1) % nb, x - 1)
            if start_next:
                g_start((j + 2) % nb, x + 2)

        pltpu.sync_copy(srcm_hbm.at[w], siall)
        for j in range(nb - 1):
            g_start(j, j)
        pltpu.sync_copy(dstm_hbm.at[w], diall)
        pltpu.sync_copy(srct_hbm.at[w], sit)
        pltpu.sync_copy(dstt_hbm.at[w], dit)
        pltpu.sync_copy(z_hbm, acc.at[pl.ds(rbase, rpt)])
        plsc.subcore_barrier()

        for j in range(nb):
            step(j, j, first=(j == 0), start_next=True)

        @pl.loop(1, nchunks // nb - 1)
        def _(t):
            x0 = t * nb
            for j in range(nb):
                step(j, x0 + j, first=False, start_next=True)

        x0 = nchunks - nb
        for j in range(nb):
            step(j, x0 + j, first=False, start_next=(x0 + j + 2 < nchunks))
        s_wait((nchunks - 1) % nb, nchunks - 1)
        tslc = pl.ds(0, tail)
        pltpu.async_copy(h_hbm.at[sit], gbufs[0].at[tslc], gsems[0]).wait()
        pltpu.async_copy(gbufs[0].at[tslc], acc.at[dit], ssems[0],
                         add=True).wait()

        plsc.subcore_barrier()
        pltpu.sync_copy(acc.at[pl.ds(rbase, rpt)],
                        out_hbm.at[pl.ds(c * N + rbase, rpt)])

    return k(h, srcm, srct, dstm, dstt, z_rows)


def _prep_body(x_ref, do0_ref, do1_ref, di0_ref, di1_ref,
               h_ref, nin_ref, nout_ref):
    dout = do0_ref[...] + do1_ref[...]
    din = di0_ref[...] + di1_ref[...]
    nout = lax.rsqrt(jnp.maximum(dout, 1.0))
    nin = lax.rsqrt(jnp.maximum(din, 1.0))
    h_ref[...] = (x_ref[...] * nout).astype(jnp.bfloat16)
    nin_ref[...] = nin
    nout_ref[...] = nout


def _tc_prep(x, do0, do1, di0, di1):
    n, d = x.shape
    grid = (n // _ROWS,)
    vec = pl.BlockSpec((_ROWS, 1), lambda i: (i, 0))
    return pl.pallas_call(
        _prep_body,
        grid=grid,
        in_specs=[pl.BlockSpec((_ROWS, d), lambda i: (i, 0)), vec, vec, vec, vec],
        out_specs=[pl.BlockSpec((_ROWS, d), lambda i: (i, 0)), vec, vec],
        out_shape=[jax.ShapeDtypeStruct((n, d), jnp.bfloat16),
                   jax.ShapeDtypeStruct((n, 1), jnp.float32),
                   jax.ShapeDtypeStruct((n, 1), jnp.float32)],
    )(x, do0, do1, di0, di1)


def _layer_body(a0_ref, a1_ref, nin_ref, nout_ref, w_ref, b_ref, h_ref):
    a = (a0_ref[...].astype(jnp.float32) + a1_ref[...].astype(jnp.float32))
    a = a * nin_ref[...]
    z = jnp.dot(a.astype(jnp.bfloat16), w_ref[...].astype(jnp.bfloat16),
                preferred_element_type=jnp.float32) + b_ref[...]
    h_ref[...] = (jax.nn.sigmoid(z) * nout_ref[...]).astype(jnp.bfloat16)


def _tc_layer(agg2, nin, nout, W, b):
    n2, d = agg2.shape
    n = n2 // 2
    grid = (n // _ROWS,)
    nblk = n // _ROWS
    vec = pl.BlockSpec((_ROWS, 1), lambda i: (i, 0))
    return pl.pallas_call(
        _layer_body,
        grid=grid,
        in_specs=[pl.BlockSpec((_ROWS, d), lambda i: (i, 0)),
                  pl.BlockSpec((_ROWS, d), lambda i: (i + nblk, 0)),
                  vec, vec,
                  pl.BlockSpec((d, d), lambda i: (0, 0)),
                  pl.BlockSpec((1, d), lambda i: (0, 0))],
        out_specs=pl.BlockSpec((_ROWS, d), lambda i: (i, 0)),
        out_shape=jax.ShapeDtypeStruct((n, d), jnp.bfloat16),
    )(agg2, agg2, nin, nout, W, b.reshape(1, d))


def _final_body(a0_ref, a1_ref, nin_ref, w_ref, b_ref, wnc_ref, bnc_ref,
                npred_ref, ysum_ref):
    i = pl.program_id(0)
    a = (a0_ref[...].astype(jnp.float32) + a1_ref[...].astype(jnp.float32))
    a = a * nin_ref[...]
    z = jnp.dot(a.astype(jnp.bfloat16), w_ref[...].astype(jnp.bfloat16),
                preferred_element_type=jnp.float32) + b_ref[...]
    y = jax.nn.sigmoid(z)
    npred_ref[...] = jax.nn.sigmoid(
        jnp.sum(y * wnc_ref[...], axis=1, keepdims=True) + bnc_ref[...])

    @pl.when(i == 0)
    def _():
        ysum_ref[...] = jnp.zeros_like(ysum_ref)

    ysum_ref[...] += jnp.sum(y, axis=0, keepdims=True)


def _tc_final(agg2, nin, W, b, wncT, bnc):
    n2, d = agg2.shape
    n = n2 // 2
    grid = (n // _ROWS,)
    nblk = n // _ROWS
    vec = pl.BlockSpec((_ROWS, 1), lambda i: (i, 0))
    full = lambda shape: pl.BlockSpec(shape, lambda i: (0, 0))
    return pl.pallas_call(
        _final_body,
        grid=grid,
        in_specs=[pl.BlockSpec((_ROWS, d), lambda i: (i, 0)),
                  pl.BlockSpec((_ROWS, d), lambda i: (i + nblk, 0)),
                  vec, full((d, d)), full((1, d)),
                  full((1, d)), full((1, 1))],
        out_specs=[vec, full((1, d))],
        out_shape=[jax.ShapeDtypeStruct((n, 1), jnp.float32),
                   jax.ShapeDtypeStruct((1, d), jnp.float32)],
    )(agg2, agg2, nin, W, b.reshape(1, d), wncT, bnc.reshape(1, 1))


def _head_body(sp_ref, ss_ref, wrop_ref, brop_ref, wros_ref, bros_ref,
               wm1_ref, bm1_ref, wm2_ref, bm2_ref, o_ref, *, n_nodes):
    inv = 1.0 / n_nodes
    bf = jnp.bfloat16
    gp = jax.nn.sigmoid(
        jnp.dot((sp_ref[...] * inv).astype(bf), wrop_ref[...].astype(bf),
                preferred_element_type=jnp.float32) + brop_ref[...])
    gs = jax.nn.sigmoid(
        jnp.dot((ss_ref[...] * inv).astype(bf), wros_ref[...].astype(bf),
                preferred_element_type=jnp.float32) + bros_ref[...])
    cat = jnp.concatenate([gp, gs], axis=1)
    z1 = jnp.dot(cat.astype(bf), wm1_ref[...].astype(bf),
                 preferred_element_type=jnp.float32) + bm1_ref[...]
    z2 = jnp.sum(z1 * wm2_ref[...], axis=1, keepdims=True) + bm2_ref[...]
    o_ref[...] = jax.nn.sigmoid(z2)


def _tc_head(sp, ss, Wro_p, bro_p, Wro_s, bro_s, Wm1, bm1, Wm2, bm2, n_nodes):
    d = sp.shape[1]
    full = lambda shape: pl.BlockSpec(shape, lambda: (0,) * len(shape))
    return pl.pallas_call(
        functools.partial(_head_body, n_nodes=n_nodes),
        in_specs=[full((1, d)), full((1, d)),
                  full((d, d)), full((1, d)),
                  full((d, d)), full((1, d)),
                  full((2 * d, d)), full((1, d)),
                  full((1, d)), full((1, 1))],
        out_specs=full((1, 1)),
        out_shape=jax.ShapeDtypeStruct((1, 1), jnp.float32),
    )(sp, ss, Wro_p, bro_p.reshape(1, d), Wro_s, bro_s.reshape(1, d),
      Wm1, bm1.reshape(1, d), Wm2.reshape(1, d), bm2.reshape(1, 1))


def _split_edges(v, e):
    ept = e // (_NC * _NS)
    nchunks = (ept // _KA) // _NBUF * _NBUF
    main = nchunks * _KA
    vt = v.reshape(_NC * _NS, ept)
    return vt[:, :main].reshape(_NC * _NS, nchunks, _KA), vt[:, main:]


def kernel(x_p, edge_index_p, x_s, edge_index_s, Wp, bp, Ws, bs,
           Wro_p, bro_p, Wro_s, bro_s, Wnc_p, bnc_p, Wnc_s, bnc_s,
           Wm1, bm1, Wm2, bm2):
    n, d = x_p.shape
    idx4 = jnp.concatenate([edge_index_p, edge_index_s], axis=0)
    z_pad = jnp.zeros((_NPAD,), jnp.float32)
    iota_pad = jnp.arange(_NPAD, dtype=jnp.int32)
    z_rows = jnp.zeros((n // _NS, d), jnp.bfloat16)

    deg = _sc_degrees(idx4, z_pad, iota_pad)
    dslc = lambda a, c: deg[a, c, :n, None]

    L = Wp.shape[0]
    e = edge_index_p.shape[1]
    spm, spt = _split_edges(edge_index_p[0], e)
    dpm, dpt = _split_edges(edge_index_p[1], e)
    ssm, sst = _split_edges(edge_index_s[0], e)
    dsm, dst_ = _split_edges(edge_index_s[1], e)
    hp, nin_p, nout_p = _tc_prep(x_p, dslc(0, 0), dslc(0, 1),
                                 dslc(1, 0), dslc(1, 1))
    hs, nin_s, nout_s = _tc_prep(x_s, dslc(2, 0), dslc(2, 1),
                                 dslc(3, 0), dslc(3, 1))
    for i in range(L - 1):
        ap = _sc_agg(hp, spm, spt, dpm, dpt, z_rows)
        a_s = _sc_agg(hs, ssm, sst, dsm, dst_, z_rows)
        hp = _tc_layer(ap, nin_p, nout_p, Wp[i], bp[i])
        hs = _tc_layer(a_s, nin_s, nout_s, Ws[i], bs[i])
    ap = _sc_agg(hp, spm, spt, dpm, dpt, z_rows)
    a_s = _sc_agg(hs, ssm, sst, dsm, dst_, z_rows)
    npred_p, ysum_p = _tc_final(ap, nin_p, Wp[L - 1], bp[L - 1],
                                Wnc_p.reshape(1, d), bnc_p)
    npred_s, ysum_s = _tc_final(a_s, nin_s, Ws[L - 1], bs[L - 1],
                                Wnc_s.reshape(1, d), bnc_s)

    dist = _tc_head(ysum_p, ysum_s, Wro_p, bro_p, Wro_s, bro_s,
                    Wm1, bm1, Wm2, bm2, n).reshape(1)
    return (dist, npred_p, npred_s)
